# Initial kernel scaffold; baseline (speedup 1.0000x reference)
import jax
import jax.numpy as jnp
from jax.experimental import pallas as pl


def kernel(S1, S2):
    raise NotImplementedError("write your pallas kernel here")


# speedup vs baseline: 1.3287x; 1.3287x over previous
import jax
import jax.numpy as jnp
from jax.experimental import pallas as pl


def _nn_kernel(s1_ref, s2t_ref, out_ref):
    x = s1_ref[0]
    yt = s2t_ref[0]
    inner = jnp.dot(x, yt, preferred_element_type=jnp.float32)
    x_sq = jnp.sum(x * x, axis=1, keepdims=True)
    y_sq = jnp.sum(yt * yt, axis=0, keepdims=True)
    dist = x_sq + y_sq - 2.0 * inner
    idx = jnp.argmin(dist, axis=1).astype(jnp.int32)
    out_ref[0, 0, :] = idx


def kernel(S1, S2):
    B, N, D = S1.shape
    M = S2.shape[1]
    TN = 256
    nb = N // TN
    S2t = jnp.transpose(S2, (0, 2, 1))

    out = pl.pallas_call(
        _nn_kernel,
        grid=(B, nb),
        in_specs=[
            pl.BlockSpec((1, TN, D), lambda b, i: (b, i, 0)),
            pl.BlockSpec((1, D, M), lambda b, i: (b, 0, 0)),
        ],
        out_specs=pl.BlockSpec((1, 1, TN), lambda b, i: (b * nb + i, 0, 0)),
        out_shape=jax.ShapeDtypeStruct((B * nb, 1, TN), jnp.int32),
    )(S1, S2t)
    return out.reshape(B, N).astype(jnp.int64)
